# Initial kernel scaffold; baseline (speedup 1.0000x reference)
import jax
import jax.numpy as jnp
from jax.experimental import pallas as pl


def kernel(coord, numbers, edge_index, rcov, r4r2, c6ab, cn_ref):
    raise NotImplementedError("write your pallas kernel here")


# speedup vs baseline: 18.7729x; 18.7729x over previous
import functools

import jax
import jax.numpy as jnp
from jax import lax
from jax.experimental import pallas as pl
from jax.experimental.pallas import tpu as pltpu
from jax.experimental.pallas import tpu_sc as plsc

S6 = 1.0
S8 = 0.7875
A1 = 0.4289
A2 = 4.4407
SMOOTH_ON = 12.0
SMOOTH_OFF = 15.0

_NW = 32
_C = 128
_L = 16


def _sqrt(x):
    m = x
    s = jnp.ones_like(x)
    for ebits in (16, 8, 4, 2, 1):
        f2 = jnp.float32(4.0 ** ebits)
        f = jnp.float32(2.0 ** ebits)
        c = m >= f2
        m = jnp.where(c, m * (1.0 / f2), m)
        s = jnp.where(c, s * f, s)
        c2 = m < 1.0
        m = jnp.where(c2, m * f2, m)
        s = jnp.where(c2, s * (1.0 / f), s)
    y = 0.5 * (1.0 + m)
    for _ in range(3):
        y = 0.5 * (y + m / y)
    return s * y


def _cos_window(r):
    t = jnp.clip((r - SMOOTH_ON) / (SMOOTH_OFF - SMOOTH_ON), 0.0, 1.0)
    x = jnp.float32(3.14159265358979) * (t - 0.5)
    x2 = x * x
    s = x * (1.0 + x2 * (-1.0 / 6.0 + x2 * (1.0 / 120.0 + x2 *
             (-1.0 / 5040.0 + x2 * (1.0 / 362880.0)))))
    return 0.5 * (1.0 - s)


def _pass_a(nchunks, nrows, idrows_n):
    ew = nchunks * _C

    def body(xh, yh, zh, numh, rcovh, r4h, iih, jjh, idrh,
             re_o, pe_o, qqe_o, cn_o,
             ib, jb, xib, yib, zib, xjb, yjb, zjb, nib, njb,
             rb, qb, pb, rcv, r4v, mycn, idr, shcn, sem):
        cid = lax.axis_index("c")
        sid = lax.axis_index("s")
        wid = sid * 2 + cid
        wstart = wid * ew

        pltpu.sync_copy(rcovh, rcv)
        pltpu.sync_copy(r4h, r4v)
        pltpu.sync_copy(idrh, idr)

        zero = jnp.zeros((_L,), jnp.float32)

        def zrow(k, carry):
            for t in range(_C // _L):
                mycn[k, pl.ds(t * _L, _L)] = zero
            return carry
        lax.fori_loop(0, nrows, zrow, 0)

        @pl.when(sid == 0)
        def _():
            pltpu.sync_copy(mycn, shcn)
        plsc.subcore_barrier()

        def chunk(c, carry):
            base = wstart + c * _C
            pltpu.sync_copy(iih.at[pl.ds(base, _C)], ib)
            pltpu.sync_copy(jjh.at[pl.ds(base, _C)], jb)
            pltpu.async_copy(xh.at[ib], xib, sem).wait()
            pltpu.async_copy(yh.at[ib], yib, sem).wait()
            pltpu.async_copy(zh.at[ib], zib, sem).wait()
            pltpu.async_copy(xh.at[jb], xjb, sem).wait()
            pltpu.async_copy(yh.at[jb], yjb, sem).wait()
            pltpu.async_copy(zh.at[jb], zjb, sem).wait()
            pltpu.async_copy(numh.at[ib], nib, sem).wait()
            pltpu.async_copy(numh.at[jb], njb, sem).wait()
            for t in range(_C // _L):
                sl = pl.ds(t * _L, _L)
                iv = ib[sl]
                jv = jb[sl]
                dx = xjb[sl] - xib[sl]
                dy = yjb[sl] - yib[sl]
                dz = zjb[sl] - zib[sl]
                r2 = jnp.maximum(dx * dx + dy * dy + dz * dz,
                                 jnp.float32(1e-12))
                r = _sqrt(r2)
                zi = nib[sl]
                zj = njb[sl]
                rc = plsc.load_gather(rcv, [zi]) + plsc.load_gather(rcv, [zj])
                cf = 1.0 / (1.0 + jnp.exp(-16.0 * (rc / r - 1.0)))
                valid = (iv != jv) & (r < SMOOTH_OFF)
                cf = jnp.where(valid, cf, 0.0)
                plsc.addupdate_scatter(
                    mycn,
                    [lax.shift_right_logical(iv, 7), iv & 127],
                    cf)
                qq = 3.0 * plsc.load_gather(r4v, [zi]) * \
                    plsc.load_gather(r4v, [zj])
                rb[sl] = r
                qb[sl] = qq
                pb[sl] = zi * 95 + zj
            pltpu.sync_copy(rb, re_o.at[pl.ds(base, _C)])
            pltpu.sync_copy(qb, qqe_o.at[pl.ds(base, _C)])
            pltpu.sync_copy(pb, pe_o.at[pl.ds(base, _C)])
            return carry
        lax.fori_loop(0, nchunks, chunk, 0)

        for k in range(idrows_n):
            pltpu.sync_copy(mycn.at[pl.ds(k * _C, _C)],
                            shcn.at[idr.at[k]], add=True)
        plsc.subcore_barrier()

        @pl.when(sid == 0)
        def _():
            pltpu.sync_copy(shcn, cn_o.at[cid])

    return body


def _pass_b(nchunks, nref, tpad):
    ew = nchunks * _C

    def body(iih, jjh, re_i, pe_i, qqe_i, cn0h, cn1h, c6t, cit, cjt,
             part_o,
             ib, jb, rb, qb, pb, a0i, a1i, a0j, a1j,
             tc6, tci, tcj, acc, sem):
        cid = lax.axis_index("c")
        sid = lax.axis_index("s")
        wid = sid * 2 + cid
        wstart = wid * ew

        acc[...] = jnp.zeros((_L,), jnp.float32)

        def chunk(c, carry):
            base = wstart + c * _C
            pltpu.sync_copy(iih.at[pl.ds(base, _C)], ib)
            pltpu.sync_copy(jjh.at[pl.ds(base, _C)], jb)
            pltpu.sync_copy(re_i.at[pl.ds(base, _C)], rb)
            pltpu.sync_copy(qqe_i.at[pl.ds(base, _C)], qb)
            pltpu.sync_copy(pe_i.at[pl.ds(base, _C)], pb)
            pltpu.async_copy(cn0h.at[ib], a0i, sem).wait()
            pltpu.async_copy(cn1h.at[ib], a1i, sem).wait()
            pltpu.async_copy(cn0h.at[jb], a0j, sem).wait()
            pltpu.async_copy(cn1h.at[jb], a1j, sem).wait()
            pltpu.async_copy(c6t.at[pb], tc6, sem).wait()
            pltpu.async_copy(cit.at[pb], tci, sem).wait()
            pltpu.async_copy(cjt.at[pb], tcj, sem).wait()
            for t in range(_C // _L):
                sl = pl.ds(t * _L, _L)
                lane = lax.iota(jnp.int32, _L) + t * _L
                iv = ib[sl]
                jv = jb[sl]
                r = rb[sl]
                qq = qb[sl]
                cni = a0i[sl] + a1i[sl]
                cnj = a0j[sl] + a1j[sl]
                zs = jnp.zeros((_L,), jnp.float32)
                nm = jnp.zeros((_L,), jnp.float32)
                for q in range(nref):
                    qv = jnp.full((_L,), q, jnp.int32)
                    av = plsc.load_gather(tci, [lane, qv])
                    bv = plsc.load_gather(tcj, [lane, qv])
                    cv = plsc.load_gather(tc6, [lane, qv])
                    da = cni - av
                    db = cnj - bv
                    w = jnp.exp(-4.0 * (da * da + db * db))
                    zs = zs + w
                    nm = nm + w * cv
                c6 = nm / jnp.maximum(zs, jnp.float32(1e-10))
                c8 = c6 * qq
                sq = jnp.full((_L,), 8.0, jnp.float32)
                for _ in range(5):
                    sq = 0.5 * (sq + qq / sq)
                rr = A1 * sq + A2
                rr2 = rr * rr
                rr6 = rr2 * rr2 * rr2
                rr8 = rr6 * rr2
                r2 = r * r
                r6 = r2 * r2 * r2
                r8 = r6 * r2
                e = -S6 * c6 / (r6 + rr6) - S8 * c8 / (r8 + rr8)
                e = e * _cos_window(r)
                valid = (iv != jv) & (r < SMOOTH_OFF)
                e = jnp.where(valid, e, 0.0)
                acc[...] = acc[...] + e
            return carry
        lax.fori_loop(0, nchunks, chunk, 0)

        pltpu.sync_copy(acc, part_o.at[wid])

    return body


def kernel(coord, numbers, edge_index, rcov, r4r2, c6ab, cn_ref):
    n = coord.shape[0]
    e = edge_index.shape[1]
    z = c6ab.shape[0]
    nref = c6ab.shape[2]
    rr = nref * nref
    tpad = 32

    nchunks = -(-e // (_NW * _C))
    ep = nchunks * _NW * _C
    nrows_raw = -(-n // _C)
    idrows_n = -(-nrows_raw // _C)
    nrows = idrows_n * _C

    ii = jnp.zeros((ep,), jnp.int32).at[:e].set(edge_index[0].astype(jnp.int32))
    jj = jnp.zeros((ep,), jnp.int32).at[:e].set(edge_index[1].astype(jnp.int32))
    xh = coord[:, 0]
    yh = coord[:, 1]
    zh = coord[:, 2]
    numh = numbers.astype(jnp.int32)
    rcovh = jnp.zeros((96,), jnp.float32).at[:z].set(rcov)
    r4h = jnp.zeros((96,), jnp.float32).at[:z].set(r4r2)
    c6t = jnp.zeros((z * z, tpad), jnp.float32).at[:, :rr].set(
        c6ab.reshape(z * z, rr))
    cit = jnp.zeros((z * z, tpad), jnp.float32).at[:, :rr].set(
        cn_ref.reshape(z * z, rr))
    cjt = jnp.zeros((z * z, tpad), jnp.float32).at[:, :rr].set(
        cn_ref.transpose(1, 0, 3, 2).reshape(z * z, rr))
    idrh = jnp.arange(nrows, dtype=jnp.int32).reshape(idrows_n, _C)

    f32 = jnp.float32
    i32 = jnp.int32
    mesh = plsc.VectorSubcoreMesh(core_axis_name="c", subcore_axis_name="s")
    cparams = pltpu.CompilerParams(needs_layout_passes=False,
                                   use_tc_tiling_on_sc=False)

    ka = pl.kernel(
        _pass_a(nchunks, nrows, idrows_n),
        out_type=(
            jax.ShapeDtypeStruct((ep,), f32),
            jax.ShapeDtypeStruct((ep,), i32),
            jax.ShapeDtypeStruct((ep,), f32),
            jax.ShapeDtypeStruct((2, nrows, _C), f32),
        ),
        mesh=mesh,
        scratch_types=[
            pltpu.VMEM((_C,), i32), pltpu.VMEM((_C,), i32),
            pltpu.VMEM((_C,), f32), pltpu.VMEM((_C,), f32),
            pltpu.VMEM((_C,), f32), pltpu.VMEM((_C,), f32),
            pltpu.VMEM((_C,), f32), pltpu.VMEM((_C,), f32),
            pltpu.VMEM((_C,), i32), pltpu.VMEM((_C,), i32),
            pltpu.VMEM((_C,), f32), pltpu.VMEM((_C,), f32),
            pltpu.VMEM((_C,), i32),
            pltpu.VMEM((96,), f32), pltpu.VMEM((96,), f32),
            pltpu.VMEM((nrows, _C), f32),
            pltpu.VMEM((idrows_n, _C), i32),
            pltpu.VMEM_SHARED((nrows, _C), f32),
            pltpu.SemaphoreType.DMA,
        ],
        compiler_params=cparams,
    )
    re_e, pe_e, qqe, cnout = ka(xh, yh, zh, numh, rcovh, r4h, ii, jj, idrh)

    cn0 = cnout[0].reshape(nrows * _C)
    cn1 = cnout[1].reshape(nrows * _C)

    kb = pl.kernel(
        _pass_b(nchunks, rr, tpad),
        out_type=jax.ShapeDtypeStruct((_NW, _L), f32),
        mesh=mesh,
        scratch_types=[
            pltpu.VMEM((_C,), i32), pltpu.VMEM((_C,), i32),
            pltpu.VMEM((_C,), f32), pltpu.VMEM((_C,), f32),
            pltpu.VMEM((_C,), i32),
            pltpu.VMEM((_C,), f32), pltpu.VMEM((_C,), f32),
            pltpu.VMEM((_C,), f32), pltpu.VMEM((_C,), f32),
            pltpu.VMEM((_C, tpad), f32), pltpu.VMEM((_C, tpad), f32),
            pltpu.VMEM((_C, tpad), f32),
            pltpu.VMEM((_L,), f32),
            pltpu.SemaphoreType.DMA,
        ],
        compiler_params=cparams,
    )
    part = kb(ii, jj, re_e, pe_e, qqe, cn0, cn1, c6t, cit, cjt)
    return 0.5 * jnp.sum(part)


# speedup vs baseline: 32.1962x; 1.7150x over previous
import functools

import jax
import jax.numpy as jnp
from jax import lax
from jax.experimental import pallas as pl
from jax.experimental.pallas import tpu as pltpu
from jax.experimental.pallas import tpu_sc as plsc

S6 = 1.0
S8 = 0.7875
A1 = 0.4289
A2 = 4.4407
SMOOTH_ON = 12.0
SMOOTH_OFF = 15.0

_NW = 32
_C = 128
_L = 16


def _sqrt(x):
    m = x
    s = jnp.ones_like(x)
    for ebits in (16, 8, 4, 2, 1):
        f2 = jnp.float32(4.0 ** ebits)
        f = jnp.float32(2.0 ** ebits)
        c = m >= f2
        m = jnp.where(c, m * (1.0 / f2), m)
        s = jnp.where(c, s * f, s)
        c2 = m < 1.0
        m = jnp.where(c2, m * f2, m)
        s = jnp.where(c2, s * (1.0 / f), s)
    y = 0.5 * (1.0 + m)
    for _ in range(3):
        y = 0.5 * (y + m / y)
    return s * y


def _cos_window(r):
    t = jnp.clip((r - SMOOTH_ON) / (SMOOTH_OFF - SMOOTH_ON), 0.0, 1.0)
    x = jnp.float32(3.14159265358979) * (t - 0.5)
    x2 = x * x
    s = x * (1.0 + x2 * (-1.0 / 6.0 + x2 * (1.0 / 120.0 + x2 *
             (-1.0 / 5040.0 + x2 * (1.0 / 362880.0)))))
    return 0.5 * (1.0 - s)


def _pass_a(nchunks, nrows, idrows_n):
    ew = nchunks * _C

    def body(xh, yh, zh, numh, rcovh, r4h, iih, jjh, idrh,
             re_o, pe_o, qqe_o, cn_o,
             ib, jb, xib, yib, zib, xjb, yjb, zjb, nib, njb,
             rb, qb, pb, rcv, r4v, mycn, idr, shcn,
             s0, s1, s2, s3, s4, s5, s6, s7):
        cid = lax.axis_index("c")
        sid = lax.axis_index("s")
        wid = sid * 2 + cid
        wstart = wid * ew

        pltpu.sync_copy(rcovh, rcv)
        pltpu.sync_copy(r4h, r4v)
        pltpu.sync_copy(idrh, idr)

        zero = jnp.zeros((_L,), jnp.float32)

        def zrow(k, carry):
            for t in range(_C // _L):
                mycn[k, pl.ds(t * _L, _L)] = zero
            return carry
        lax.fori_loop(0, nrows, zrow, 0)

        @pl.when(sid == 0)
        def _():
            pltpu.sync_copy(mycn, shcn)
        plsc.subcore_barrier()

        def chunk(c, carry):
            base = wstart + c * _C
            pltpu.sync_copy(iih.at[pl.ds(base, _C)], ib)
            pltpu.sync_copy(jjh.at[pl.ds(base, _C)], jb)
            cps = [
                pltpu.async_copy(xh.at[ib], xib, s0),
                pltpu.async_copy(yh.at[ib], yib, s1),
                pltpu.async_copy(zh.at[ib], zib, s2),
                pltpu.async_copy(xh.at[jb], xjb, s3),
                pltpu.async_copy(yh.at[jb], yjb, s4),
                pltpu.async_copy(zh.at[jb], zjb, s5),
                pltpu.async_copy(numh.at[ib], nib, s6),
                pltpu.async_copy(numh.at[jb], njb, s7),
            ]
            for cp in cps:
                cp.wait()
            for t in range(_C // _L):
                sl = pl.ds(t * _L, _L)
                iv = ib[sl]
                jv = jb[sl]
                dx = xjb[sl] - xib[sl]
                dy = yjb[sl] - yib[sl]
                dz = zjb[sl] - zib[sl]
                r2 = jnp.maximum(dx * dx + dy * dy + dz * dz,
                                 jnp.float32(1e-12))
                r = _sqrt(r2)
                zi = nib[sl]
                zj = njb[sl]
                rc = plsc.load_gather(rcv, [zi]) + plsc.load_gather(rcv, [zj])
                cf = 1.0 / (1.0 + jnp.exp(-16.0 * (rc / r - 1.0)))
                valid = (iv != jv) & (r < SMOOTH_OFF)
                cf = jnp.where(valid, cf, 0.0)
                plsc.addupdate_scatter(
                    mycn,
                    [lax.shift_right_logical(iv, 7), iv & 127],
                    cf)
                qq = 3.0 * plsc.load_gather(r4v, [zi]) * \
                    plsc.load_gather(r4v, [zj])
                rb[sl] = r
                qb[sl] = qq
                pb[sl] = zi * 95 + zj
            pltpu.sync_copy(rb, re_o.at[pl.ds(base, _C)])
            pltpu.sync_copy(qb, qqe_o.at[pl.ds(base, _C)])
            pltpu.sync_copy(pb, pe_o.at[pl.ds(base, _C)])
            return carry
        lax.fori_loop(0, nchunks, chunk, 0)

        for k in range(idrows_n):
            pltpu.sync_copy(mycn.at[pl.ds(k * _C, _C)],
                            shcn.at[idr.at[k]], add=True)
        plsc.subcore_barrier()

        @pl.when(sid == 0)
        def _():
            pltpu.sync_copy(shcn, cn_o.at[cid])

    return body


def _pass_b(nchunks, nref, tpad):
    ew = nchunks * _C

    def body(iih, jjh, re_i, pe_i, qqe_i, cn0h, cn1h, c6t, cit, cjt,
             part_o,
             ib, jb, rb, qb, pb, a0i, a1i, a0j, a1j,
             tc6, tci, tcj, acc, s0, s1, s2, s3, s4, s5, s6):
        cid = lax.axis_index("c")
        sid = lax.axis_index("s")
        wid = sid * 2 + cid
        wstart = wid * ew

        acc[...] = jnp.zeros((_L,), jnp.float32)

        def chunk(c, carry):
            base = wstart + c * _C
            pltpu.sync_copy(iih.at[pl.ds(base, _C)], ib)
            pltpu.sync_copy(jjh.at[pl.ds(base, _C)], jb)
            pltpu.sync_copy(re_i.at[pl.ds(base, _C)], rb)
            pltpu.sync_copy(qqe_i.at[pl.ds(base, _C)], qb)
            pltpu.sync_copy(pe_i.at[pl.ds(base, _C)], pb)
            cps = [
                pltpu.async_copy(cn0h.at[ib], a0i, s0),
                pltpu.async_copy(cn1h.at[ib], a1i, s1),
                pltpu.async_copy(cn0h.at[jb], a0j, s2),
                pltpu.async_copy(cn1h.at[jb], a1j, s3),
                pltpu.async_copy(c6t.at[pb], tc6, s4),
                pltpu.async_copy(cit.at[pb], tci, s5),
                pltpu.async_copy(cjt.at[pb], tcj, s6),
            ]
            for cp in cps:
                cp.wait()
            for t in range(_C // _L):
                sl = pl.ds(t * _L, _L)
                lane = lax.iota(jnp.int32, _L) + t * _L
                iv = ib[sl]
                jv = jb[sl]
                r = rb[sl]
                qq = qb[sl]
                cni = a0i[sl] + a1i[sl]
                cnj = a0j[sl] + a1j[sl]
                zs = jnp.zeros((_L,), jnp.float32)
                nm = jnp.zeros((_L,), jnp.float32)
                for q in range(nref):
                    qv = jnp.full((_L,), q, jnp.int32)
                    av = plsc.load_gather(tci, [lane, qv])
                    bv = plsc.load_gather(tcj, [lane, qv])
                    cv = plsc.load_gather(tc6, [lane, qv])
                    da = cni - av
                    db = cnj - bv
                    w = jnp.exp(-4.0 * (da * da + db * db))
                    zs = zs + w
                    nm = nm + w * cv
                c6 = nm / jnp.maximum(zs, jnp.float32(1e-10))
                c8 = c6 * qq
                sq = jnp.full((_L,), 8.0, jnp.float32)
                for _ in range(5):
                    sq = 0.5 * (sq + qq / sq)
                rr = A1 * sq + A2
                rr2 = rr * rr
                rr6 = rr2 * rr2 * rr2
                rr8 = rr6 * rr2
                r2 = r * r
                r6 = r2 * r2 * r2
                r8 = r6 * r2
                e = -S6 * c6 / (r6 + rr6) - S8 * c8 / (r8 + rr8)
                e = e * _cos_window(r)
                valid = (iv != jv) & (r < SMOOTH_OFF)
                e = jnp.where(valid, e, 0.0)
                acc[...] = acc[...] + e
            return carry
        lax.fori_loop(0, nchunks, chunk, 0)

        pltpu.sync_copy(acc, part_o.at[wid])

    return body


def kernel(coord, numbers, edge_index, rcov, r4r2, c6ab, cn_ref):
    n = coord.shape[0]
    e = edge_index.shape[1]
    z = c6ab.shape[0]
    nref = c6ab.shape[2]
    rr = nref * nref
    tpad = 32

    nchunks = -(-e // (_NW * _C))
    ep = nchunks * _NW * _C
    nrows_raw = -(-n // _C)
    idrows_n = -(-nrows_raw // _C)
    nrows = idrows_n * _C

    ii = jnp.zeros((ep,), jnp.int32).at[:e].set(edge_index[0].astype(jnp.int32))
    jj = jnp.zeros((ep,), jnp.int32).at[:e].set(edge_index[1].astype(jnp.int32))
    xh = coord[:, 0]
    yh = coord[:, 1]
    zh = coord[:, 2]
    numh = numbers.astype(jnp.int32)
    rcovh = jnp.zeros((96,), jnp.float32).at[:z].set(rcov)
    r4h = jnp.zeros((96,), jnp.float32).at[:z].set(r4r2)
    c6t = jnp.zeros((z * z, tpad), jnp.float32).at[:, :rr].set(
        c6ab.reshape(z * z, rr))
    cit = jnp.zeros((z * z, tpad), jnp.float32).at[:, :rr].set(
        cn_ref.reshape(z * z, rr))
    cjt = jnp.zeros((z * z, tpad), jnp.float32).at[:, :rr].set(
        cn_ref.transpose(1, 0, 3, 2).reshape(z * z, rr))
    idrh = jnp.arange(nrows, dtype=jnp.int32).reshape(idrows_n, _C)

    f32 = jnp.float32
    i32 = jnp.int32
    mesh = plsc.VectorSubcoreMesh(core_axis_name="c", subcore_axis_name="s")
    cparams = pltpu.CompilerParams(needs_layout_passes=False,
                                   use_tc_tiling_on_sc=False)

    ka = pl.kernel(
        _pass_a(nchunks, nrows, idrows_n),
        out_type=(
            jax.ShapeDtypeStruct((ep,), f32),
            jax.ShapeDtypeStruct((ep,), i32),
            jax.ShapeDtypeStruct((ep,), f32),
            jax.ShapeDtypeStruct((2, nrows, _C), f32),
        ),
        mesh=mesh,
        scratch_types=[
            pltpu.VMEM((_C,), i32), pltpu.VMEM((_C,), i32),
            pltpu.VMEM((_C,), f32), pltpu.VMEM((_C,), f32),
            pltpu.VMEM((_C,), f32), pltpu.VMEM((_C,), f32),
            pltpu.VMEM((_C,), f32), pltpu.VMEM((_C,), f32),
            pltpu.VMEM((_C,), i32), pltpu.VMEM((_C,), i32),
            pltpu.VMEM((_C,), f32), pltpu.VMEM((_C,), f32),
            pltpu.VMEM((_C,), i32),
            pltpu.VMEM((96,), f32), pltpu.VMEM((96,), f32),
            pltpu.VMEM((nrows, _C), f32),
            pltpu.VMEM((idrows_n, _C), i32),
            pltpu.VMEM_SHARED((nrows, _C), f32),
        ] + [pltpu.SemaphoreType.DMA] * 8,
        compiler_params=cparams,
    )
    re_e, pe_e, qqe, cnout = ka(xh, yh, zh, numh, rcovh, r4h, ii, jj, idrh)

    cn0 = cnout[0].reshape(nrows * _C)
    cn1 = cnout[1].reshape(nrows * _C)

    kb = pl.kernel(
        _pass_b(nchunks, rr, tpad),
        out_type=jax.ShapeDtypeStruct((_NW, _L), f32),
        mesh=mesh,
        scratch_types=[
            pltpu.VMEM((_C,), i32), pltpu.VMEM((_C,), i32),
            pltpu.VMEM((_C,), f32), pltpu.VMEM((_C,), f32),
            pltpu.VMEM((_C,), i32),
            pltpu.VMEM((_C,), f32), pltpu.VMEM((_C,), f32),
            pltpu.VMEM((_C,), f32), pltpu.VMEM((_C,), f32),
            pltpu.VMEM((_C, tpad), f32), pltpu.VMEM((_C, tpad), f32),
            pltpu.VMEM((_C, tpad), f32),
            pltpu.VMEM((_L,), f32),
        ] + [pltpu.SemaphoreType.DMA] * 7,
        compiler_params=cparams,
    )
    part = kb(ii, jj, re_e, pe_e, qqe, cn0, cn1, c6t, cit, cjt)
    return 0.5 * jnp.sum(part)
